# Initial kernel scaffold; baseline (speedup 1.0000x reference)
import jax
import jax.numpy as jnp
from jax.experimental import pallas as pl


def kernel(x, edge_index, edge_attr, W1a, b1a, W1b, b1b, root1, bias1, bn1_g, bn1_b, W2a, b2a, W2b, b2b, root2, bias2, bn2_g, bn2_b, Wfc1, bfc1, Wfc2, bfc2):
    raise NotImplementedError("write your pallas kernel here")


# speedup vs baseline: 2.4589x; 2.4589x over previous
import functools

import jax
import jax.numpy as jnp
from jax import lax
from jax.experimental import pallas as pl
from jax.experimental.pallas import tpu as pltpu
from jax.experimental.pallas import tpu_sc as plsc

N = 20000
DN = 16
DE = 8
NC = 2
NS = 16
NW = NC * NS
CHUNK = 128
E_PAD = 81920
NCH = E_PAD // (NW * CHUNK)
GG = 2
N_PAD = 20096
RPS = N_PAD // NS
BE = 2048
LW = 128


def _elu(v):
    return jnp.where(v > 0, v, jnp.exp(v) - 1.0)


def _sc_mesh():
    return plsc.VectorSubcoreMesh(
        core_axis_name="c", subcore_axis_name="s",
        num_cores=NC, num_subcores=NS)


def _sc_gather(table, idx):
    def body(table_hbm, idx_hbm, out_hbm, idx_v, buf0, buf1,
             sem_g0, sem_g1, sem_w):
        wid = lax.axis_index("s") * NC + lax.axis_index("c")
        base = wid * NCH * CHUNK
        pltpu.sync_copy(idx_hbm.at[pl.ds(base, NCH * CHUNK)], idx_v)

        bufs = [buf0, buf1]
        sems = [sem_g0, sem_g1]
        ng = NCH // GG

        def fire(g):
            buf = bufs[g % 2]
            return [
                pltpu.async_copy(
                    table_hbm.at[idx_v.at[pl.ds((g * GG + k) * CHUNK, CHUNK)]],
                    buf.at[pl.ds(k * CHUNK, CHUNK)], sems[g % 2])
                for k in range(GG)
            ]

        wbs = [None, None]
        pend = fire(0)
        for g in range(ng):
            if wbs[(g + 1) % 2] is not None:
                wbs[(g + 1) % 2].wait()
                wbs[(g + 1) % 2] = None
            nxt = fire(g + 1) if g + 1 < ng else None
            for dsc in pend:
                dsc.wait()
            wbs[g % 2] = pltpu.async_copy(
                bufs[g % 2],
                out_hbm.at[pl.ds(base + g * GG * CHUNK, GG * CHUNK)], sem_w)
            pend = nxt
        for wb in wbs:
            if wb is not None:
                wb.wait()

    return pl.kernel(
        body,
        out_type=jax.ShapeDtypeStruct((E_PAD, LW), jnp.float32),
        mesh=_sc_mesh(),
        scratch_types=[
            pltpu.VMEM((NCH * CHUNK,), jnp.int32),
            pltpu.VMEM((GG * CHUNK, LW), jnp.float32),
            pltpu.VMEM((GG * CHUNK, LW), jnp.float32),
            pltpu.SemaphoreType.DMA,
            pltpu.SemaphoreType.DMA,
            pltpu.SemaphoreType.DMA,
        ],
        compiler_params=pltpu.CompilerParams(use_tc_tiling_on_sc=False),
    )(table, idx)


def _sc_scatter_add(msg, idx, dv):
    zeros = jnp.zeros((N_PAD, dv), jnp.float32)

    def body(msg_hbm, idx_hbm, zero_hbm, out_hbm, idx_v0, idx_v1, msg_v0,
             msg_v1, acc_sh, sem_i0, sem_i1, sem_m0, sem_m1):
        c = lax.axis_index("c")
        s = lax.axis_index("s")
        wid = s * NC + c
        base = wid * NCH * CHUNK

        pltpu.sync_copy(zero_hbm.at[pl.ds(s * RPS, RPS)],
                        acc_sh.at[pl.ds(s * RPS, RPS)])
        plsc.subcore_barrier()

        bufs = [(idx_v0, msg_v0, sem_i0, sem_m0),
                (idx_v1, msg_v1, sem_i1, sem_m1)]

        def fire(j):
            iv, mv, si, sm = bufs[j % 2]
            di = pltpu.async_copy(idx_hbm.at[pl.ds(base + j * CHUNK, CHUNK)],
                                  iv, si)
            dm = pltpu.async_copy(
                msg_hbm.at[pl.ds(base + j * CHUNK, CHUNK), pl.ds(0, dv)],
                mv, sm)
            return di, dm

        pend = fire(0)
        for j in range(NCH):
            nxt = fire(j + 1) if j + 1 < NCH else None
            pend[0].wait()
            pend[1].wait()
            iv, mv = bufs[j % 2][0], bufs[j % 2][1]
            pltpu.sync_copy(mv, acc_sh.at[iv], add=True)
            pend = nxt

        plsc.subcore_barrier()
        pltpu.sync_copy(acc_sh.at[pl.ds(s * RPS, RPS)],
                        out_hbm.at[c, pl.ds(s * RPS, RPS), pl.ds(0, dv)])

    return pl.kernel(
        body,
        out_type=jax.ShapeDtypeStruct((NC, N_PAD, LW), jnp.float32),
        mesh=_sc_mesh(),
        scratch_types=[
            pltpu.VMEM((CHUNK,), jnp.int32),
            pltpu.VMEM((CHUNK,), jnp.int32),
            pltpu.VMEM((CHUNK, dv), jnp.float32),
            pltpu.VMEM((CHUNK, dv), jnp.float32),
            pltpu.VMEM_SHARED((N_PAD, dv), jnp.float32),
            pltpu.SemaphoreType.DMA,
            pltpu.SemaphoreType.DMA,
            pltpu.SemaphoreType.DMA,
            pltpu.SemaphoreType.DMA,
        ],
        compiler_params=pltpu.CompilerParams(use_tc_tiling_on_sc=False),
    )(msg, idx, zeros)


def _bn_apply(v, st_ref, g_ref, b_ref):
    mu = st_ref[0:1, :] * (1.0 / N)
    var = st_ref[1:2, :] * (1.0 / N) - mu * mu
    return _elu(g_ref[...] * (v - mu) * lax.rsqrt(var + 1e-5) + b_ref[...])


def _edge_body(cin, ones_cols, ea_ref, xs_ref, wa_ref, ba_ref, r_ref, t_ref,
               wb_ref, bb_ref, *rest):
    if len(rest) == 4:
        st_ref, g_ref, b_ref, out_ref = rest
        xs = _bn_apply(xs_ref[:, :cin], st_ref, g_ref, b_ref)
    else:
        (out_ref,) = rest
        xs = xs_ref[:, :cin]
    h = jnp.maximum(
        jnp.dot(ea_ref[...], wa_ref[...], preferred_element_type=jnp.float32)
        + ba_ref[...], 0.0)
    z = (jnp.dot(h, r_ref[...], preferred_element_type=jnp.float32)
         * jnp.dot(xs, t_ref[...], preferred_element_type=jnp.float32))
    msg = (jnp.dot(z, wb_ref[...], preferred_element_type=jnp.float32)
           + jnp.dot(xs, bb_ref[...], preferred_element_type=jnp.float32))
    if ones_cols:
        msg = jnp.concatenate(
            [msg, jnp.ones((msg.shape[0], ones_cols), jnp.float32)], axis=1)
    out_ref[:, :msg.shape[1]] = msg


def _edge_call(ea, xs, wa, ba, rmat, tmat, wb, bb, cin, ones_cols, norm=None):
    body = functools.partial(_edge_body, cin, ones_cols)
    args = [ea, xs, wa, ba, rmat, tmat, wb, bb]
    if norm is not None:
        args.extend(norm)
    return pl.pallas_call(
        body,
        grid=(E_PAD // BE,),
        in_specs=[
            pl.BlockSpec((BE, DE), lambda i: (i, 0)),
            pl.BlockSpec((BE, LW), lambda i: (i, 0)),
        ] + [pl.BlockSpec(a.shape, lambda i: (0, 0)) for a in args[2:]],
        out_specs=pl.BlockSpec((BE, LW), lambda i: (i, 0)),
        out_shape=jax.ShapeDtypeStruct((E_PAD, LW), jnp.float32),
    )(*args)


BN_BLK = 2000


def _pre1_body(part_ref, x_ref, root_ref, bias_ref, e1_ref, st_ref):
    i = pl.program_id(0)
    tot = part_ref[0, :, :48] + part_ref[1, :, :48]
    ssum = tot[:, :32]
    cnt = tot[:, 32:33]
    rc = 1.0 / jnp.maximum(cnt, 1.0)
    a = (ssum * rc
         + jnp.dot(x_ref[...], root_ref[...], preferred_element_type=jnp.float32)
         + bias_ref[...])
    e1 = _elu(a)
    e1_ref[:, :33] = jnp.concatenate([e1, rc], axis=1)

    @pl.when(i == 0)
    def _():
        st_ref[...] = jnp.zeros_like(st_ref)

    st_ref[0:1, :] += jnp.sum(e1, axis=0, keepdims=True)
    st_ref[1:2, :] += jnp.sum(e1 * e1, axis=0, keepdims=True)


def _pre1_call(part, x, root, bias):
    return pl.pallas_call(
        _pre1_body,
        grid=(N // BN_BLK,),
        in_specs=[
            pl.BlockSpec((2, BN_BLK, LW), lambda i: (0, i, 0)),
            pl.BlockSpec((BN_BLK, DN), lambda i: (i, 0)),
            pl.BlockSpec((DN, 32), lambda i: (0, 0)),
            pl.BlockSpec((1, 32), lambda i: (0, 0)),
        ],
        out_specs=(pl.BlockSpec((BN_BLK, LW), lambda i: (i, 0)),
                   pl.BlockSpec((8, 32), lambda i: (0, 0))),
        out_shape=(jax.ShapeDtypeStruct((N, LW), jnp.float32),
                   jax.ShapeDtypeStruct((8, 32), jnp.float32)),
    )(part, x, root, bias)


def _pre2_body(part_ref, e1_ref, st1_ref, g1_ref, b1_ref, root_ref,
               bias_ref, e2_ref, st_ref):
    i = pl.program_id(0)
    tot = part_ref[0, :, :64] + part_ref[1, :, :64]
    h1 = _bn_apply(e1_ref[:, :32], st1_ref, g1_ref, b1_ref)
    rc = e1_ref[:, 32:33]
    a = (tot * rc
         + jnp.dot(h1, root_ref[...], preferred_element_type=jnp.float32)
         + bias_ref[...])
    e2 = _elu(a)
    e2_ref[...] = e2

    @pl.when(i == 0)
    def _():
        st_ref[...] = jnp.zeros_like(st_ref)

    st_ref[0:1, :] += jnp.sum(e2, axis=0, keepdims=True)
    st_ref[1:2, :] += jnp.sum(e2 * e2, axis=0, keepdims=True)


def _pre2_call(part, e1, st1, g1, b1, root, bias):
    return pl.pallas_call(
        _pre2_body,
        grid=(N // BN_BLK,),
        in_specs=[
            pl.BlockSpec((2, BN_BLK, LW), lambda i: (0, i, 0)),
            pl.BlockSpec((BN_BLK, LW), lambda i: (i, 0)),
            pl.BlockSpec((8, 32), lambda i: (0, 0)),
            pl.BlockSpec((1, 32), lambda i: (0, 0)),
            pl.BlockSpec((1, 32), lambda i: (0, 0)),
            pl.BlockSpec((32, 64), lambda i: (0, 0)),
            pl.BlockSpec((1, 64), lambda i: (0, 0)),
        ],
        out_specs=(pl.BlockSpec((BN_BLK, 64), lambda i: (i, 0)),
                   pl.BlockSpec((8, 64), lambda i: (0, 0))),
        out_shape=(jax.ShapeDtypeStruct((N, 64), jnp.float32),
                   jax.ShapeDtypeStruct((8, 64), jnp.float32)),
    )(part, e1, st1, g1, b1, root, bias)


def _head_body(e2_ref, st_ref, g_ref, b_ref, wf1_ref, bf1_ref, wf2_ref,
               bf2_ref, out_ref):
    i = pl.program_id(0)
    h2 = _bn_apply(e2_ref[...], st_ref, g_ref, b_ref)
    f1 = _elu(jnp.dot(h2, wf1_ref[...], preferred_element_type=jnp.float32)
              + bf1_ref[...])
    f2 = _elu(jnp.dot(f1, wf2_ref[...], preferred_element_type=jnp.float32)
              + bf2_ref[...])

    @pl.when(i == 0)
    def _():
        out_ref[...] = jnp.zeros_like(out_ref)

    out_ref[...] += jnp.sum(f2).reshape(1, 1)


def _head_call(e2, st, g, b, wf1, bf1, wf2, bf2):
    return pl.pallas_call(
        _head_body,
        grid=(N // BN_BLK,),
        in_specs=[
            pl.BlockSpec((BN_BLK, 64), lambda i: (i, 0)),
            pl.BlockSpec((8, 64), lambda i: (0, 0)),
            pl.BlockSpec((1, 64), lambda i: (0, 0)),
            pl.BlockSpec((1, 64), lambda i: (0, 0)),
            pl.BlockSpec((64, 128), lambda i: (0, 0)),
            pl.BlockSpec((1, 128), lambda i: (0, 0)),
            pl.BlockSpec((128, 1), lambda i: (0, 0)),
            pl.BlockSpec((1, 1), lambda i: (0, 0)),
        ],
        out_specs=pl.BlockSpec((1, 1), lambda i: (0, 0)),
        out_shape=jax.ShapeDtypeStruct((1, 1), jnp.float32),
    )(e2, st, g, b, wf1, bf1, wf2, bf2)


def kernel(x, edge_index, edge_attr, W1a, b1a, W1b, b1b, root1, bias1,
           bn1_g, bn1_b, W2a, b2a, W2b, b2b, root2, bias2, bn2_g, bn2_b,
           Wfc1, bfc1, Wfc2, bfc2):
    e = edge_index.shape[1]
    src = edge_index[0]
    dst = edge_index[1]

    src_p = jnp.concatenate([src, jnp.zeros((E_PAD - e,), jnp.int32)])
    dst_p = jnp.concatenate([dst, jnp.full((E_PAD - e,), N, jnp.int32)])

    nh = W1a.shape[1]
    wb1 = W1b.reshape(nh * DN, 32)
    wb2 = W2b.reshape(nh * 32, 64)
    r1 = jnp.repeat(jnp.eye(nh, dtype=jnp.float32), DN, axis=1)
    t1 = jnp.tile(jnp.eye(DN, dtype=jnp.float32), (1, nh))
    r2 = jnp.repeat(jnp.eye(nh, dtype=jnp.float32), 32, axis=1)
    t2 = jnp.tile(jnp.eye(32, dtype=jnp.float32), (1, nh))

    x128 = jnp.pad(x, ((0, 0), (0, LW - DN)))

    xs1 = _sc_gather(x128, src_p)
    msg1 = _edge_call(edge_attr, xs1, W1a, b1a.reshape(1, -1), r1, t1, wb1,
                      b1b.reshape(DN, 32), DN, ones_cols=16)
    part1 = _sc_scatter_add(msg1, dst_p, 48)
    e1, st1 = _pre1_call(part1, x, root1, bias1.reshape(1, -1))
    g1 = bn1_g.reshape(1, -1)
    bb1 = bn1_b.reshape(1, -1)

    xs2 = _sc_gather(e1, src_p)
    msg2 = _edge_call(edge_attr, xs2, W2a, b2a.reshape(1, -1), r2, t2, wb2,
                      b2b.reshape(32, 64), 32, ones_cols=0,
                      norm=(st1, g1, bb1))
    part2 = _sc_scatter_add(msg2, dst_p, 64)

    e2, st2 = _pre2_call(part2, e1, st1, g1, bb1, root2,
                         bias2.reshape(1, -1))
    out = _head_call(e2, st2, bn2_g.reshape(1, -1), bn2_b.reshape(1, -1),
                     Wfc1, bfc1.reshape(1, -1), Wfc2, bfc2.reshape(1, 1))
    return out.reshape(1)


# speedup vs baseline: 3.3569x; 1.3652x over previous
import functools

import jax
import jax.numpy as jnp
from jax import lax
from jax.experimental import pallas as pl
from jax.experimental.pallas import tpu as pltpu
from jax.experimental.pallas import tpu_sc as plsc

N = 20000
DN = 16
DE = 8
NC = 2
NS = 16
NW = NC * NS
CHUNK = 128
E_PAD = 81920
NCH = E_PAD // (NW * CHUNK)
GG = 2
N_PAD = 20096
RPS = N_PAD // NS
BE = 2048
LW = 128


def _elu(v):
    return jnp.where(v > 0, v, jnp.exp(v) - 1.0)


def _sc_mesh():
    return plsc.VectorSubcoreMesh(
        core_axis_name="c", subcore_axis_name="s",
        num_cores=NC, num_subcores=NS)


def _sc_gather(table, idx, dv):
    def body(table_hbm, idx_hbm, out_hbm, idx_v, buf0, buf1,
             sem_g0, sem_g1, sem_w):
        wid = lax.axis_index("s") * NC + lax.axis_index("c")
        base = wid * NCH * CHUNK
        pltpu.sync_copy(idx_hbm.at[pl.ds(base, NCH * CHUNK)], idx_v)

        bufs = [buf0, buf1]
        sems = [sem_g0, sem_g1]
        ng = NCH // GG

        def fire(g):
            buf = bufs[g % 2]
            return [
                pltpu.async_copy(
                    table_hbm.at[idx_v.at[pl.ds((g * GG + k) * CHUNK, CHUNK)]],
                    buf.at[pl.ds(k * CHUNK, CHUNK)], sems[g % 2])
                for k in range(GG)
            ]

        wbs = [None, None]
        pend = fire(0)
        for g in range(ng):
            if wbs[(g + 1) % 2] is not None:
                wbs[(g + 1) % 2].wait()
                wbs[(g + 1) % 2] = None
            nxt = fire(g + 1) if g + 1 < ng else None
            for dsc in pend:
                dsc.wait()
            wbs[g % 2] = pltpu.async_copy(
                bufs[g % 2],
                out_hbm.at[pl.ds(base + g * GG * CHUNK, GG * CHUNK),
                           pl.ds(0, dv)], sem_w)
            pend = nxt
        for wb in wbs:
            if wb is not None:
                wb.wait()

    return pl.kernel(
        body,
        out_type=jax.ShapeDtypeStruct((E_PAD, LW), jnp.float32),
        mesh=_sc_mesh(),
        scratch_types=[
            pltpu.VMEM((NCH * CHUNK,), jnp.int32),
            pltpu.VMEM((GG * CHUNK, dv), jnp.float32),
            pltpu.VMEM((GG * CHUNK, dv), jnp.float32),
            pltpu.SemaphoreType.DMA,
            pltpu.SemaphoreType.DMA,
            pltpu.SemaphoreType.DMA,
        ],
        compiler_params=pltpu.CompilerParams(use_tc_tiling_on_sc=False),
    )(table, idx)


def _sc_scatter_add(msg, idx, dv):
    zeros = jnp.zeros((N_PAD, dv), jnp.float32)

    def body(msg_hbm, idx_hbm, zero_hbm, out_hbm, idx_v0, idx_v1, msg_v0,
             msg_v1, acc_sh, sem_i0, sem_i1, sem_m0, sem_m1):
        c = lax.axis_index("c")
        s = lax.axis_index("s")
        wid = s * NC + c
        base = wid * NCH * CHUNK

        pltpu.sync_copy(zero_hbm.at[pl.ds(s * RPS, RPS)],
                        acc_sh.at[pl.ds(s * RPS, RPS)])
        plsc.subcore_barrier()

        bufs = [(idx_v0, msg_v0, sem_i0, sem_m0),
                (idx_v1, msg_v1, sem_i1, sem_m1)]

        def fire(j):
            iv, mv, si, sm = bufs[j % 2]
            di = pltpu.async_copy(idx_hbm.at[pl.ds(base + j * CHUNK, CHUNK)],
                                  iv, si)
            dm = pltpu.async_copy(
                msg_hbm.at[pl.ds(base + j * CHUNK, CHUNK), pl.ds(0, dv)],
                mv, sm)
            return di, dm

        pend = fire(0)
        for j in range(NCH):
            nxt = fire(j + 1) if j + 1 < NCH else None
            pend[0].wait()
            pend[1].wait()
            iv, mv = bufs[j % 2][0], bufs[j % 2][1]
            pltpu.sync_copy(mv, acc_sh.at[iv], add=True)
            pend = nxt

        plsc.subcore_barrier()
        pltpu.sync_copy(acc_sh.at[pl.ds(s * RPS, RPS)],
                        out_hbm.at[c, pl.ds(s * RPS, RPS), pl.ds(0, dv)])

    return pl.kernel(
        body,
        out_type=jax.ShapeDtypeStruct((NC, N_PAD, LW), jnp.float32),
        mesh=_sc_mesh(),
        scratch_types=[
            pltpu.VMEM((CHUNK,), jnp.int32),
            pltpu.VMEM((CHUNK,), jnp.int32),
            pltpu.VMEM((CHUNK, dv), jnp.float32),
            pltpu.VMEM((CHUNK, dv), jnp.float32),
            pltpu.VMEM_SHARED((N_PAD, dv), jnp.float32),
            pltpu.SemaphoreType.DMA,
            pltpu.SemaphoreType.DMA,
            pltpu.SemaphoreType.DMA,
            pltpu.SemaphoreType.DMA,
        ],
        compiler_params=pltpu.CompilerParams(use_tc_tiling_on_sc=False),
    )(msg, idx, zeros)


def _bn_apply(v, st_ref, g_ref, b_ref):
    mu = st_ref[0:1, :] * (1.0 / N)
    var = st_ref[1:2, :] * (1.0 / N) - mu * mu
    return _elu(g_ref[...] * (v - mu) * lax.rsqrt(var + 1e-5) + b_ref[...])


def _edge_body(cin, ones_cols, ea_ref, xs_ref, wa_ref, ba_ref, r_ref, t_ref,
               wb_ref, bb_ref, *rest):
    if len(rest) == 4:
        st_ref, g_ref, b_ref, out_ref = rest
        xs = _bn_apply(xs_ref[:, :cin], st_ref, g_ref, b_ref)
    else:
        (out_ref,) = rest
        xs = xs_ref[:, :cin]
    h = jnp.maximum(
        jnp.dot(ea_ref[...], wa_ref[...], preferred_element_type=jnp.float32)
        + ba_ref[...], 0.0)
    z = (jnp.dot(h, r_ref[...], preferred_element_type=jnp.float32)
         * jnp.dot(xs, t_ref[...], preferred_element_type=jnp.float32))
    msg = (jnp.dot(z, wb_ref[...], preferred_element_type=jnp.float32)
           + jnp.dot(xs, bb_ref[...], preferred_element_type=jnp.float32))
    if ones_cols:
        msg = jnp.concatenate(
            [msg, jnp.ones((msg.shape[0], ones_cols), jnp.float32)], axis=1)
    out_ref[:, :msg.shape[1]] = msg


def _edge_call(ea, xs, wa, ba, rmat, tmat, wb, bb, cin, ones_cols, norm=None):
    body = functools.partial(_edge_body, cin, ones_cols)
    args = [ea, xs, wa, ba, rmat, tmat, wb, bb]
    if norm is not None:
        args.extend(norm)
    return pl.pallas_call(
        body,
        grid=(E_PAD // BE,),
        in_specs=[
            pl.BlockSpec((BE, DE), lambda i: (i, 0)),
            pl.BlockSpec((BE, LW), lambda i: (i, 0)),
        ] + [pl.BlockSpec(a.shape, lambda i: (0, 0)) for a in args[2:]],
        out_specs=pl.BlockSpec((BE, LW), lambda i: (i, 0)),
        out_shape=jax.ShapeDtypeStruct((E_PAD, LW), jnp.float32),
    )(*args)


BN_BLK = 2000


def _pre1_body(part_ref, x_ref, root_ref, bias_ref, e1_ref, st_ref):
    i = pl.program_id(0)
    tot = part_ref[0, :, :48] + part_ref[1, :, :48]
    ssum = tot[:, :32]
    cnt = tot[:, 32:33]
    rc = 1.0 / jnp.maximum(cnt, 1.0)
    a = (ssum * rc
         + jnp.dot(x_ref[...], root_ref[...], preferred_element_type=jnp.float32)
         + bias_ref[...])
    e1 = _elu(a)
    e1_ref[:, :33] = jnp.concatenate([e1, rc], axis=1)

    @pl.when(i == 0)
    def _():
        st_ref[...] = jnp.zeros_like(st_ref)

    st_ref[0:1, :] += jnp.sum(e1, axis=0, keepdims=True)
    st_ref[1:2, :] += jnp.sum(e1 * e1, axis=0, keepdims=True)


def _pre1_call(part, x, root, bias):
    return pl.pallas_call(
        _pre1_body,
        grid=(N // BN_BLK,),
        in_specs=[
            pl.BlockSpec((2, BN_BLK, LW), lambda i: (0, i, 0)),
            pl.BlockSpec((BN_BLK, DN), lambda i: (i, 0)),
            pl.BlockSpec((DN, 32), lambda i: (0, 0)),
            pl.BlockSpec((1, 32), lambda i: (0, 0)),
        ],
        out_specs=(pl.BlockSpec((BN_BLK, 48), lambda i: (i, 0)),
                   pl.BlockSpec((8, 32), lambda i: (0, 0))),
        out_shape=(jax.ShapeDtypeStruct((N, 48), jnp.float32),
                   jax.ShapeDtypeStruct((8, 32), jnp.float32)),
    )(part, x, root, bias)


def _pre2_body(part_ref, e1_ref, st1_ref, g1_ref, b1_ref, root_ref,
               bias_ref, e2_ref, st_ref):
    i = pl.program_id(0)
    tot = part_ref[0, :, :64] + part_ref[1, :, :64]
    h1 = _bn_apply(e1_ref[:, :32], st1_ref, g1_ref, b1_ref)
    rc = e1_ref[:, 32:33]
    a = (tot * rc
         + jnp.dot(h1, root_ref[...], preferred_element_type=jnp.float32)
         + bias_ref[...])
    e2 = _elu(a)
    e2_ref[...] = e2

    @pl.when(i == 0)
    def _():
        st_ref[...] = jnp.zeros_like(st_ref)

    st_ref[0:1, :] += jnp.sum(e2, axis=0, keepdims=True)
    st_ref[1:2, :] += jnp.sum(e2 * e2, axis=0, keepdims=True)


def _pre2_call(part, e1, st1, g1, b1, root, bias):
    return pl.pallas_call(
        _pre2_body,
        grid=(N // BN_BLK,),
        in_specs=[
            pl.BlockSpec((2, BN_BLK, LW), lambda i: (0, i, 0)),
            pl.BlockSpec((BN_BLK, 48), lambda i: (i, 0)),
            pl.BlockSpec((8, 32), lambda i: (0, 0)),
            pl.BlockSpec((1, 32), lambda i: (0, 0)),
            pl.BlockSpec((1, 32), lambda i: (0, 0)),
            pl.BlockSpec((32, 64), lambda i: (0, 0)),
            pl.BlockSpec((1, 64), lambda i: (0, 0)),
        ],
        out_specs=(pl.BlockSpec((BN_BLK, 64), lambda i: (i, 0)),
                   pl.BlockSpec((8, 64), lambda i: (0, 0))),
        out_shape=(jax.ShapeDtypeStruct((N, 64), jnp.float32),
                   jax.ShapeDtypeStruct((8, 64), jnp.float32)),
    )(part, e1, st1, g1, b1, root, bias)


def _head_body(e2_ref, st_ref, g_ref, b_ref, wf1_ref, bf1_ref, wf2_ref,
               bf2_ref, out_ref):
    i = pl.program_id(0)
    h2 = _bn_apply(e2_ref[...], st_ref, g_ref, b_ref)
    f1 = _elu(jnp.dot(h2, wf1_ref[...], preferred_element_type=jnp.float32)
              + bf1_ref[...])
    f2 = _elu(jnp.dot(f1, wf2_ref[...], preferred_element_type=jnp.float32)
              + bf2_ref[...])

    @pl.when(i == 0)
    def _():
        out_ref[...] = jnp.zeros_like(out_ref)

    out_ref[...] += jnp.sum(f2).reshape(1, 1)


def _head_call(e2, st, g, b, wf1, bf1, wf2, bf2):
    return pl.pallas_call(
        _head_body,
        grid=(N // BN_BLK,),
        in_specs=[
            pl.BlockSpec((BN_BLK, 64), lambda i: (i, 0)),
            pl.BlockSpec((8, 64), lambda i: (0, 0)),
            pl.BlockSpec((1, 64), lambda i: (0, 0)),
            pl.BlockSpec((1, 64), lambda i: (0, 0)),
            pl.BlockSpec((64, 128), lambda i: (0, 0)),
            pl.BlockSpec((1, 128), lambda i: (0, 0)),
            pl.BlockSpec((128, 1), lambda i: (0, 0)),
            pl.BlockSpec((1, 1), lambda i: (0, 0)),
        ],
        out_specs=pl.BlockSpec((1, 1), lambda i: (0, 0)),
        out_shape=jax.ShapeDtypeStruct((1, 1), jnp.float32),
    )(e2, st, g, b, wf1, bf1, wf2, bf2)


def kernel(x, edge_index, edge_attr, W1a, b1a, W1b, b1b, root1, bias1,
           bn1_g, bn1_b, W2a, b2a, W2b, b2b, root2, bias2, bn2_g, bn2_b,
           Wfc1, bfc1, Wfc2, bfc2):
    e = edge_index.shape[1]
    src = edge_index[0]
    dst = edge_index[1]

    src_p = jnp.concatenate([src, jnp.zeros((E_PAD - e,), jnp.int32)])
    dst_p = jnp.concatenate([dst, jnp.full((E_PAD - e,), N, jnp.int32)])

    nh = W1a.shape[1]
    wb1 = W1b.reshape(nh * DN, 32)
    wb2 = W2b.reshape(nh * 32, 64)
    r1 = jnp.repeat(jnp.eye(nh, dtype=jnp.float32), DN, axis=1)
    t1 = jnp.tile(jnp.eye(DN, dtype=jnp.float32), (1, nh))
    r2 = jnp.repeat(jnp.eye(nh, dtype=jnp.float32), 32, axis=1)
    t2 = jnp.tile(jnp.eye(32, dtype=jnp.float32), (1, nh))

    xs1 = _sc_gather(x, src_p, DN)
    msg1 = _edge_call(edge_attr, xs1, W1a, b1a.reshape(1, -1), r1, t1, wb1,
                      b1b.reshape(DN, 32), DN, ones_cols=16)
    part1 = _sc_scatter_add(msg1, dst_p, 48)
    e1, st1 = _pre1_call(part1, x, root1, bias1.reshape(1, -1))
    g1 = bn1_g.reshape(1, -1)
    bb1 = bn1_b.reshape(1, -1)

    xs2 = _sc_gather(e1, src_p, 48)
    msg2 = _edge_call(edge_attr, xs2, W2a, b2a.reshape(1, -1), r2, t2, wb2,
                      b2b.reshape(32, 64), 32, ones_cols=0,
                      norm=(st1, g1, bb1))
    part2 = _sc_scatter_add(msg2, dst_p, 64)

    e2, st2 = _pre2_call(part2, e1, st1, g1, bb1, root2,
                         bias2.reshape(1, -1))
    out = _head_call(e2, st2, bn2_g.reshape(1, -1), bn2_b.reshape(1, -1),
                     Wfc1, bfc1.reshape(1, -1), Wfc2, bfc2.reshape(1, 1))
    return out.reshape(1)


# speedup vs baseline: 3.4294x; 1.0216x over previous
import functools

import jax
import jax.numpy as jnp
from jax import lax
from jax.experimental import pallas as pl
from jax.experimental.pallas import tpu as pltpu
from jax.experimental.pallas import tpu_sc as plsc

N = 20000
DN = 16
DE = 8
NC = 2
NS = 16
NW = NC * NS
CHUNK = 128
E_PAD = 81920
NCH = E_PAD // (NW * CHUNK)
GG = 2
N_PAD = 20096
RPS = N_PAD // NS
BE = 4096
LW = 128


def _elu(v):
    return jnp.where(v > 0, v, jnp.exp(v) - 1.0)


def _sc_mesh():
    return plsc.VectorSubcoreMesh(
        core_axis_name="c", subcore_axis_name="s",
        num_cores=NC, num_subcores=NS)


def _sc_gather(table, idx, dv):
    def body(table_hbm, idx_hbm, out_hbm, idx_v, buf0, buf1,
             sem_g0, sem_g1, sem_w):
        wid = lax.axis_index("s") * NC + lax.axis_index("c")
        base = wid * NCH * CHUNK
        pltpu.sync_copy(idx_hbm.at[pl.ds(base, NCH * CHUNK)], idx_v)

        bufs = [buf0, buf1]
        sems = [sem_g0, sem_g1]
        ng = NCH // GG

        def fire(g):
            buf = bufs[g % 2]
            return [
                pltpu.async_copy(
                    table_hbm.at[idx_v.at[pl.ds((g * GG + k) * CHUNK, CHUNK)]],
                    buf.at[pl.ds(k * CHUNK, CHUNK)], sems[g % 2])
                for k in range(GG)
            ]

        wbs = [None, None]
        pend = fire(0)
        for g in range(ng):
            if wbs[(g + 1) % 2] is not None:
                wbs[(g + 1) % 2].wait()
                wbs[(g + 1) % 2] = None
            nxt = fire(g + 1) if g + 1 < ng else None
            for dsc in pend:
                dsc.wait()
            wbs[g % 2] = pltpu.async_copy(
                bufs[g % 2],
                out_hbm.at[pl.ds(base + g * GG * CHUNK, GG * CHUNK),
                           pl.ds(0, dv)], sem_w)
            pend = nxt
        for wb in wbs:
            if wb is not None:
                wb.wait()

    return pl.kernel(
        body,
        out_type=jax.ShapeDtypeStruct((E_PAD, LW), jnp.float32),
        mesh=_sc_mesh(),
        scratch_types=[
            pltpu.VMEM((NCH * CHUNK,), jnp.int32),
            pltpu.VMEM((GG * CHUNK, dv), jnp.float32),
            pltpu.VMEM((GG * CHUNK, dv), jnp.float32),
            pltpu.SemaphoreType.DMA,
            pltpu.SemaphoreType.DMA,
            pltpu.SemaphoreType.DMA,
        ],
        compiler_params=pltpu.CompilerParams(use_tc_tiling_on_sc=False),
    )(table, idx)


def _sc_scatter_add(msg, idx, dv):
    zeros = jnp.zeros((N_PAD, dv), jnp.float32)

    def body(msg_hbm, idx_hbm, zero_hbm, out_hbm, idx_v0, idx_v1, msg_v0,
             msg_v1, acc_sh, sem_i0, sem_i1, sem_m0, sem_m1):
        c = lax.axis_index("c")
        s = lax.axis_index("s")
        wid = s * NC + c
        base = wid * NCH * CHUNK

        pltpu.sync_copy(zero_hbm.at[pl.ds(s * RPS, RPS)],
                        acc_sh.at[pl.ds(s * RPS, RPS)])
        plsc.subcore_barrier()

        bufs = [(idx_v0, msg_v0, sem_i0, sem_m0),
                (idx_v1, msg_v1, sem_i1, sem_m1)]

        def fire(j):
            iv, mv, si, sm = bufs[j % 2]
            di = pltpu.async_copy(idx_hbm.at[pl.ds(base + j * CHUNK, CHUNK)],
                                  iv, si)
            dm = pltpu.async_copy(
                msg_hbm.at[pl.ds(base + j * CHUNK, CHUNK), pl.ds(0, dv)],
                mv, sm)
            return di, dm

        pend = fire(0)
        for j in range(NCH):
            nxt = fire(j + 1) if j + 1 < NCH else None
            pend[0].wait()
            pend[1].wait()
            iv, mv = bufs[j % 2][0], bufs[j % 2][1]
            pltpu.sync_copy(mv, acc_sh.at[iv], add=True)
            pend = nxt

        plsc.subcore_barrier()
        pltpu.sync_copy(acc_sh.at[pl.ds(s * RPS, RPS)],
                        out_hbm.at[c, pl.ds(s * RPS, RPS), pl.ds(0, dv)])

    return pl.kernel(
        body,
        out_type=jax.ShapeDtypeStruct((NC, N_PAD, LW), jnp.float32),
        mesh=_sc_mesh(),
        scratch_types=[
            pltpu.VMEM((CHUNK,), jnp.int32),
            pltpu.VMEM((CHUNK,), jnp.int32),
            pltpu.VMEM((CHUNK, dv), jnp.float32),
            pltpu.VMEM((CHUNK, dv), jnp.float32),
            pltpu.VMEM_SHARED((N_PAD, dv), jnp.float32),
            pltpu.SemaphoreType.DMA,
            pltpu.SemaphoreType.DMA,
            pltpu.SemaphoreType.DMA,
            pltpu.SemaphoreType.DMA,
        ],
        compiler_params=pltpu.CompilerParams(use_tc_tiling_on_sc=False),
    )(msg, idx, zeros)


def _bn_apply(v, st_ref, g_ref, b_ref):
    mu = st_ref[0:1, :] * (1.0 / N)
    var = st_ref[1:2, :] * (1.0 / N) - mu * mu
    return _elu(g_ref[...] * (v - mu) * lax.rsqrt(var + 1e-5) + b_ref[...])


def _edge_body(cin, ones_cols, ea_ref, xs_ref, wa_ref, ba_ref, r_ref, t_ref,
               wb_ref, bb_ref, *rest):
    if len(rest) == 4:
        st_ref, g_ref, b_ref, out_ref = rest
        xs = _bn_apply(xs_ref[:, :cin], st_ref, g_ref, b_ref)
    else:
        (out_ref,) = rest
        xs = xs_ref[:, :cin]
    h = jnp.maximum(
        jnp.dot(ea_ref[...], wa_ref[...], preferred_element_type=jnp.float32)
        + ba_ref[...], 0.0)
    z = (jnp.dot(h, r_ref[...], preferred_element_type=jnp.float32)
         * jnp.dot(xs, t_ref[...], preferred_element_type=jnp.float32))
    msg = (jnp.dot(z, wb_ref[...], preferred_element_type=jnp.float32)
           + jnp.dot(xs, bb_ref[...], preferred_element_type=jnp.float32))
    if ones_cols:
        msg = jnp.concatenate(
            [msg, jnp.ones((msg.shape[0], ones_cols), jnp.float32)], axis=1)
    out_ref[:, :msg.shape[1]] = msg


def _edge_call(ea, xs, wa, ba, rmat, tmat, wb, bb, cin, ones_cols, norm=None):
    body = functools.partial(_edge_body, cin, ones_cols)
    args = [ea, xs, wa, ba, rmat, tmat, wb, bb]
    if norm is not None:
        args.extend(norm)
    return pl.pallas_call(
        body,
        grid=(E_PAD // BE,),
        in_specs=[
            pl.BlockSpec((BE, DE), lambda i: (i, 0)),
            pl.BlockSpec((BE, LW), lambda i: (i, 0)),
        ] + [pl.BlockSpec(a.shape, lambda i: (0, 0)) for a in args[2:]],
        out_specs=pl.BlockSpec((BE, LW), lambda i: (i, 0)),
        out_shape=jax.ShapeDtypeStruct((E_PAD, LW), jnp.float32),
    )(*args)


BN_BLK = 2000


def _pre1_body(part_ref, x_ref, root_ref, bias_ref, e1_ref, st_ref):
    i = pl.program_id(0)
    tot = part_ref[0, :, :48] + part_ref[1, :, :48]
    ssum = tot[:, :32]
    cnt = tot[:, 32:33]
    rc = 1.0 / jnp.maximum(cnt, 1.0)
    a = (ssum * rc
         + jnp.dot(x_ref[...], root_ref[...], preferred_element_type=jnp.float32)
         + bias_ref[...])
    e1 = _elu(a)
    e1_ref[:, :33] = jnp.concatenate([e1, rc], axis=1)

    @pl.when(i == 0)
    def _():
        st_ref[...] = jnp.zeros_like(st_ref)

    st_ref[0:1, :] += jnp.sum(e1, axis=0, keepdims=True)
    st_ref[1:2, :] += jnp.sum(e1 * e1, axis=0, keepdims=True)


def _pre1_call(part, x, root, bias):
    return pl.pallas_call(
        _pre1_body,
        grid=(N // BN_BLK,),
        in_specs=[
            pl.BlockSpec((2, BN_BLK, LW), lambda i: (0, i, 0)),
            pl.BlockSpec((BN_BLK, DN), lambda i: (i, 0)),
            pl.BlockSpec((DN, 32), lambda i: (0, 0)),
            pl.BlockSpec((1, 32), lambda i: (0, 0)),
        ],
        out_specs=(pl.BlockSpec((BN_BLK, 48), lambda i: (i, 0)),
                   pl.BlockSpec((8, 32), lambda i: (0, 0))),
        out_shape=(jax.ShapeDtypeStruct((N, 48), jnp.float32),
                   jax.ShapeDtypeStruct((8, 32), jnp.float32)),
    )(part, x, root, bias)


def _pre2_body(part_ref, e1_ref, st1_ref, g1_ref, b1_ref, root_ref,
               bias_ref, e2_ref, st_ref):
    i = pl.program_id(0)
    tot = part_ref[0, :, :64] + part_ref[1, :, :64]
    h1 = _bn_apply(e1_ref[:, :32], st1_ref, g1_ref, b1_ref)
    rc = e1_ref[:, 32:33]
    a = (tot * rc
         + jnp.dot(h1, root_ref[...], preferred_element_type=jnp.float32)
         + bias_ref[...])
    e2 = _elu(a)
    e2_ref[...] = e2

    @pl.when(i == 0)
    def _():
        st_ref[...] = jnp.zeros_like(st_ref)

    st_ref[0:1, :] += jnp.sum(e2, axis=0, keepdims=True)
    st_ref[1:2, :] += jnp.sum(e2 * e2, axis=0, keepdims=True)


def _pre2_call(part, e1, st1, g1, b1, root, bias):
    return pl.pallas_call(
        _pre2_body,
        grid=(N // BN_BLK,),
        in_specs=[
            pl.BlockSpec((2, BN_BLK, LW), lambda i: (0, i, 0)),
            pl.BlockSpec((BN_BLK, 48), lambda i: (i, 0)),
            pl.BlockSpec((8, 32), lambda i: (0, 0)),
            pl.BlockSpec((1, 32), lambda i: (0, 0)),
            pl.BlockSpec((1, 32), lambda i: (0, 0)),
            pl.BlockSpec((32, 64), lambda i: (0, 0)),
            pl.BlockSpec((1, 64), lambda i: (0, 0)),
        ],
        out_specs=(pl.BlockSpec((BN_BLK, 64), lambda i: (i, 0)),
                   pl.BlockSpec((8, 64), lambda i: (0, 0))),
        out_shape=(jax.ShapeDtypeStruct((N, 64), jnp.float32),
                   jax.ShapeDtypeStruct((8, 64), jnp.float32)),
    )(part, e1, st1, g1, b1, root, bias)


def _head_body(e2_ref, st_ref, g_ref, b_ref, wf1_ref, bf1_ref, wf2_ref,
               bf2_ref, out_ref):
    i = pl.program_id(0)
    h2 = _bn_apply(e2_ref[...], st_ref, g_ref, b_ref)
    f1 = _elu(jnp.dot(h2, wf1_ref[...], preferred_element_type=jnp.float32)
              + bf1_ref[...])
    f2 = _elu(jnp.dot(f1, wf2_ref[...], preferred_element_type=jnp.float32)
              + bf2_ref[...])

    @pl.when(i == 0)
    def _():
        out_ref[...] = jnp.zeros_like(out_ref)

    out_ref[...] += jnp.sum(f2).reshape(1, 1)


def _head_call(e2, st, g, b, wf1, bf1, wf2, bf2):
    return pl.pallas_call(
        _head_body,
        grid=(N // BN_BLK,),
        in_specs=[
            pl.BlockSpec((BN_BLK, 64), lambda i: (i, 0)),
            pl.BlockSpec((8, 64), lambda i: (0, 0)),
            pl.BlockSpec((1, 64), lambda i: (0, 0)),
            pl.BlockSpec((1, 64), lambda i: (0, 0)),
            pl.BlockSpec((64, 128), lambda i: (0, 0)),
            pl.BlockSpec((1, 128), lambda i: (0, 0)),
            pl.BlockSpec((128, 1), lambda i: (0, 0)),
            pl.BlockSpec((1, 1), lambda i: (0, 0)),
        ],
        out_specs=pl.BlockSpec((1, 1), lambda i: (0, 0)),
        out_shape=jax.ShapeDtypeStruct((1, 1), jnp.float32),
    )(e2, st, g, b, wf1, bf1, wf2, bf2)


def kernel(x, edge_index, edge_attr, W1a, b1a, W1b, b1b, root1, bias1,
           bn1_g, bn1_b, W2a, b2a, W2b, b2b, root2, bias2, bn2_g, bn2_b,
           Wfc1, bfc1, Wfc2, bfc2):
    e = edge_index.shape[1]
    src = edge_index[0]
    dst = edge_index[1]

    src_p = jnp.concatenate([src, jnp.zeros((E_PAD - e,), jnp.int32)])
    dst_p = jnp.concatenate([dst, jnp.full((E_PAD - e,), N, jnp.int32)])

    nh = W1a.shape[1]
    wb1 = W1b.reshape(nh * DN, 32)
    wb2 = W2b.reshape(nh * 32, 64)
    r1 = jnp.repeat(jnp.eye(nh, dtype=jnp.float32), DN, axis=1)
    t1 = jnp.tile(jnp.eye(DN, dtype=jnp.float32), (1, nh))
    r2 = jnp.repeat(jnp.eye(nh, dtype=jnp.float32), 32, axis=1)
    t2 = jnp.tile(jnp.eye(32, dtype=jnp.float32), (1, nh))

    xs1 = _sc_gather(x, src_p, DN)
    msg1 = _edge_call(edge_attr, xs1, W1a, b1a.reshape(1, -1), r1, t1, wb1,
                      b1b.reshape(DN, 32), DN, ones_cols=16)
    part1 = _sc_scatter_add(msg1, dst_p, 48)
    e1, st1 = _pre1_call(part1, x, root1, bias1.reshape(1, -1))
    g1 = bn1_g.reshape(1, -1)
    bb1 = bn1_b.reshape(1, -1)

    xs2 = _sc_gather(e1, src_p, 48)
    msg2 = _edge_call(edge_attr, xs2, W2a, b2a.reshape(1, -1), r2, t2, wb2,
                      b2b.reshape(32, 64), 32, ones_cols=0,
                      norm=(st1, g1, bb1))
    part2 = _sc_scatter_add(msg2, dst_p, 64)

    e2, st2 = _pre2_call(part2, e1, st1, g1, bb1, root2,
                         bias2.reshape(1, -1))
    out = _head_call(e2, st2, bn2_g.reshape(1, -1), bn2_b.reshape(1, -1),
                     Wfc1, bfc1.reshape(1, -1), Wfc2, bfc2.reshape(1, 1))
    return out.reshape(1)


# speedup vs baseline: 3.5139x; 1.0247x over previous
import functools

import jax
import jax.numpy as jnp
from jax import lax
from jax.experimental import pallas as pl
from jax.experimental.pallas import tpu as pltpu
from jax.experimental.pallas import tpu_sc as plsc

N = 20000
DN = 16
DE = 8
NC = 2
NS = 16
NW = NC * NS
CHUNK = 128
E_PAD = 81920
NCH = E_PAD // (NW * CHUNK)
GG = 2
N_PAD = 20096
RPS = N_PAD // NS
BE = 4096
LW = 128


def _elu(v):
    return jnp.where(v > 0, v, jnp.exp(v) - 1.0)


def _sc_mesh():
    return plsc.VectorSubcoreMesh(
        core_axis_name="c", subcore_axis_name="s",
        num_cores=NC, num_subcores=NS)


def _sc_gather(table, idx, dv, half):
    nch = NCH // 2
    e_half = E_PAD // 2

    def body(table_hbm, idx_hbm, out_hbm, idx_v, buf0, buf1,
             sem_g0, sem_g1, sem_w):
        wid = lax.axis_index("s") * NC + lax.axis_index("c")
        hbase = half * e_half + wid * nch * CHUNK
        base = wid * nch * CHUNK
        pltpu.sync_copy(idx_hbm.at[pl.ds(hbase, nch * CHUNK)], idx_v)

        bufs = [buf0, buf1]
        sems = [sem_g0, sem_g1]
        ng = nch // GG

        def fire(g):
            buf = bufs[g % 2]
            return [
                pltpu.async_copy(
                    table_hbm.at[idx_v.at[pl.ds((g * GG + k) * CHUNK, CHUNK)]],
                    buf.at[pl.ds(k * CHUNK, CHUNK)], sems[g % 2])
                for k in range(GG)
            ]

        wbs = [None, None]
        pend = fire(0)
        for g in range(ng):
            if wbs[(g + 1) % 2] is not None:
                wbs[(g + 1) % 2].wait()
                wbs[(g + 1) % 2] = None
            nxt = fire(g + 1) if g + 1 < ng else None
            for dsc in pend:
                dsc.wait()
            wbs[g % 2] = pltpu.async_copy(
                bufs[g % 2],
                out_hbm.at[pl.ds(base + g * GG * CHUNK, GG * CHUNK),
                           pl.ds(0, dv)], sem_w)
            pend = nxt
        for wb in wbs:
            if wb is not None:
                wb.wait()

    return pl.kernel(
        body,
        out_type=jax.ShapeDtypeStruct((e_half, LW), jnp.float32),
        mesh=_sc_mesh(),
        scratch_types=[
            pltpu.VMEM((nch * CHUNK,), jnp.int32),
            pltpu.VMEM((GG * CHUNK, dv), jnp.float32),
            pltpu.VMEM((GG * CHUNK, dv), jnp.float32),
            pltpu.SemaphoreType.DMA,
            pltpu.SemaphoreType.DMA,
            pltpu.SemaphoreType.DMA,
        ],
        compiler_params=pltpu.CompilerParams(use_tc_tiling_on_sc=False),
    )(table, idx)


def _sc_scatter_add(msg, idx, dv, half):
    zeros = jnp.zeros((N_PAD, dv), jnp.float32)
    nch = NCH // 2
    e_half = E_PAD // 2

    def body(msg_hbm, idx_hbm, zero_hbm, out_hbm, idx_v0, idx_v1, msg_v0,
             msg_v1, acc_sh, sem_i0, sem_i1, sem_m0, sem_m1):
        c = lax.axis_index("c")
        s = lax.axis_index("s")
        wid = s * NC + c
        base = wid * nch * CHUNK
        ibase = half * e_half + base

        pltpu.sync_copy(zero_hbm.at[pl.ds(s * RPS, RPS)],
                        acc_sh.at[pl.ds(s * RPS, RPS)])
        plsc.subcore_barrier()

        bufs = [(idx_v0, msg_v0, sem_i0, sem_m0),
                (idx_v1, msg_v1, sem_i1, sem_m1)]

        def fire(j):
            iv, mv, si, sm = bufs[j % 2]
            di = pltpu.async_copy(idx_hbm.at[pl.ds(ibase + j * CHUNK, CHUNK)],
                                  iv, si)
            dm = pltpu.async_copy(
                msg_hbm.at[pl.ds(base + j * CHUNK, CHUNK), pl.ds(0, dv)],
                mv, sm)
            return di, dm

        pend = fire(0)
        for j in range(nch):
            nxt = fire(j + 1) if j + 1 < nch else None
            pend[0].wait()
            pend[1].wait()
            iv, mv = bufs[j % 2][0], bufs[j % 2][1]
            pltpu.sync_copy(mv, acc_sh.at[iv], add=True)
            pend = nxt

        plsc.subcore_barrier()
        pltpu.sync_copy(acc_sh.at[pl.ds(s * RPS, RPS)],
                        out_hbm.at[c, pl.ds(s * RPS, RPS), pl.ds(0, dv)])

    return pl.kernel(
        body,
        out_type=jax.ShapeDtypeStruct((NC, N_PAD, LW), jnp.float32),
        mesh=_sc_mesh(),
        scratch_types=[
            pltpu.VMEM((CHUNK,), jnp.int32),
            pltpu.VMEM((CHUNK,), jnp.int32),
            pltpu.VMEM((CHUNK, dv), jnp.float32),
            pltpu.VMEM((CHUNK, dv), jnp.float32),
            pltpu.VMEM_SHARED((N_PAD, dv), jnp.float32),
            pltpu.SemaphoreType.DMA,
            pltpu.SemaphoreType.DMA,
            pltpu.SemaphoreType.DMA,
            pltpu.SemaphoreType.DMA,
        ],
        compiler_params=pltpu.CompilerParams(use_tc_tiling_on_sc=False),
    )(msg, idx, zeros)


def _bn_apply(v, st_ref, g_ref, b_ref):
    mu = st_ref[0:1, :] * (1.0 / N)
    var = st_ref[1:2, :] * (1.0 / N) - mu * mu
    return _elu(g_ref[...] * (v - mu) * lax.rsqrt(var + 1e-5) + b_ref[...])


def _edge_body(cin, ones_cols, ea_ref, xs_ref, wa_ref, ba_ref, r_ref, t_ref,
               wb_ref, bb_ref, *rest):
    if len(rest) == 4:
        st_ref, g_ref, b_ref, out_ref = rest
        xs = _bn_apply(xs_ref[:, :cin], st_ref, g_ref, b_ref)
    else:
        (out_ref,) = rest
        xs = xs_ref[:, :cin]
    h = jnp.maximum(
        jnp.dot(ea_ref[...], wa_ref[...], preferred_element_type=jnp.float32)
        + ba_ref[...], 0.0)
    z = (jnp.dot(h, r_ref[...], preferred_element_type=jnp.float32)
         * jnp.dot(xs, t_ref[...], preferred_element_type=jnp.float32))
    msg = (jnp.dot(z, wb_ref[...], preferred_element_type=jnp.float32)
           + jnp.dot(xs, bb_ref[...], preferred_element_type=jnp.float32))
    if ones_cols:
        msg = jnp.concatenate(
            [msg, jnp.ones((msg.shape[0], ones_cols), jnp.float32)], axis=1)
    out_ref[:, :msg.shape[1]] = msg


def _edge_call(ea, xs, wa, ba, rmat, tmat, wb, bb, cin, ones_cols, half,
               norm=None):
    body = functools.partial(_edge_body, cin, ones_cols)
    args = [ea, xs, wa, ba, rmat, tmat, wb, bb]
    if norm is not None:
        args.extend(norm)
    e_half = E_PAD // 2
    nbh = e_half // BE
    off = half * nbh
    return pl.pallas_call(
        body,
        grid=(nbh,),
        in_specs=[
            pl.BlockSpec((BE, DE), lambda i: (i + off, 0)),
            pl.BlockSpec((BE, LW), lambda i: (i, 0)),
        ] + [pl.BlockSpec(a.shape, lambda i: (0, 0)) for a in args[2:]],
        out_specs=pl.BlockSpec((BE, LW), lambda i: (i, 0)),
        out_shape=jax.ShapeDtypeStruct((e_half, LW), jnp.float32),
    )(*args)


BN_BLK = 2000


def _pre1_body(parta_ref, partb_ref, x_ref, root_ref, bias_ref, e1_ref,
               st_ref):
    i = pl.program_id(0)
    tot = (parta_ref[0, :, :48] + parta_ref[1, :, :48]
           + partb_ref[0, :, :48] + partb_ref[1, :, :48])
    ssum = tot[:, :32]
    cnt = tot[:, 32:33]
    rc = 1.0 / jnp.maximum(cnt, 1.0)
    a = (ssum * rc
         + jnp.dot(x_ref[...], root_ref[...], preferred_element_type=jnp.float32)
         + bias_ref[...])
    e1 = _elu(a)
    e1_ref[:, :33] = jnp.concatenate([e1, rc], axis=1)

    @pl.when(i == 0)
    def _():
        st_ref[...] = jnp.zeros_like(st_ref)

    st_ref[0:1, :] += jnp.sum(e1, axis=0, keepdims=True)
    st_ref[1:2, :] += jnp.sum(e1 * e1, axis=0, keepdims=True)


def _pre1_call(parta, partb, x, root, bias):
    return pl.pallas_call(
        _pre1_body,
        grid=(N // BN_BLK,),
        in_specs=[
            pl.BlockSpec((2, BN_BLK, LW), lambda i: (0, i, 0)),
            pl.BlockSpec((2, BN_BLK, LW), lambda i: (0, i, 0)),
            pl.BlockSpec((BN_BLK, DN), lambda i: (i, 0)),
            pl.BlockSpec((DN, 32), lambda i: (0, 0)),
            pl.BlockSpec((1, 32), lambda i: (0, 0)),
        ],
        out_specs=(pl.BlockSpec((BN_BLK, 48), lambda i: (i, 0)),
                   pl.BlockSpec((8, 32), lambda i: (0, 0))),
        out_shape=(jax.ShapeDtypeStruct((N, 48), jnp.float32),
                   jax.ShapeDtypeStruct((8, 32), jnp.float32)),
    )(parta, partb, x, root, bias)


def _pre2_body(parta_ref, partb_ref, e1_ref, st1_ref, g1_ref, b1_ref,
               root_ref, bias_ref, e2_ref, st_ref):
    i = pl.program_id(0)
    tot = (parta_ref[0, :, :64] + parta_ref[1, :, :64]
           + partb_ref[0, :, :64] + partb_ref[1, :, :64])
    h1 = _bn_apply(e1_ref[:, :32], st1_ref, g1_ref, b1_ref)
    rc = e1_ref[:, 32:33]
    a = (tot * rc
         + jnp.dot(h1, root_ref[...], preferred_element_type=jnp.float32)
         + bias_ref[...])
    e2 = _elu(a)
    e2_ref[...] = e2

    @pl.when(i == 0)
    def _():
        st_ref[...] = jnp.zeros_like(st_ref)

    st_ref[0:1, :] += jnp.sum(e2, axis=0, keepdims=True)
    st_ref[1:2, :] += jnp.sum(e2 * e2, axis=0, keepdims=True)


def _pre2_call(parta, partb, e1, st1, g1, b1, root, bias):
    return pl.pallas_call(
        _pre2_body,
        grid=(N // BN_BLK,),
        in_specs=[
            pl.BlockSpec((2, BN_BLK, LW), lambda i: (0, i, 0)),
            pl.BlockSpec((2, BN_BLK, LW), lambda i: (0, i, 0)),
            pl.BlockSpec((BN_BLK, 48), lambda i: (i, 0)),
            pl.BlockSpec((8, 32), lambda i: (0, 0)),
            pl.BlockSpec((1, 32), lambda i: (0, 0)),
            pl.BlockSpec((1, 32), lambda i: (0, 0)),
            pl.BlockSpec((32, 64), lambda i: (0, 0)),
            pl.BlockSpec((1, 64), lambda i: (0, 0)),
        ],
        out_specs=(pl.BlockSpec((BN_BLK, 64), lambda i: (i, 0)),
                   pl.BlockSpec((8, 64), lambda i: (0, 0))),
        out_shape=(jax.ShapeDtypeStruct((N, 64), jnp.float32),
                   jax.ShapeDtypeStruct((8, 64), jnp.float32)),
    )(parta, partb, e1, st1, g1, b1, root, bias)


def _head_body(e2_ref, st_ref, g_ref, b_ref, wf1_ref, bf1_ref, wf2_ref,
               bf2_ref, out_ref):
    i = pl.program_id(0)
    h2 = _bn_apply(e2_ref[...], st_ref, g_ref, b_ref)
    f1 = _elu(jnp.dot(h2, wf1_ref[...], preferred_element_type=jnp.float32)
              + bf1_ref[...])
    f2 = _elu(jnp.dot(f1, wf2_ref[...], preferred_element_type=jnp.float32)
              + bf2_ref[...])

    @pl.when(i == 0)
    def _():
        out_ref[...] = jnp.zeros_like(out_ref)

    out_ref[...] += jnp.sum(f2).reshape(1, 1)


def _head_call(e2, st, g, b, wf1, bf1, wf2, bf2):
    return pl.pallas_call(
        _head_body,
        grid=(N // BN_BLK,),
        in_specs=[
            pl.BlockSpec((BN_BLK, 64), lambda i: (i, 0)),
            pl.BlockSpec((8, 64), lambda i: (0, 0)),
            pl.BlockSpec((1, 64), lambda i: (0, 0)),
            pl.BlockSpec((1, 64), lambda i: (0, 0)),
            pl.BlockSpec((64, 128), lambda i: (0, 0)),
            pl.BlockSpec((1, 128), lambda i: (0, 0)),
            pl.BlockSpec((128, 1), lambda i: (0, 0)),
            pl.BlockSpec((1, 1), lambda i: (0, 0)),
        ],
        out_specs=pl.BlockSpec((1, 1), lambda i: (0, 0)),
        out_shape=jax.ShapeDtypeStruct((1, 1), jnp.float32),
    )(e2, st, g, b, wf1, bf1, wf2, bf2)


def kernel(x, edge_index, edge_attr, W1a, b1a, W1b, b1b, root1, bias1,
           bn1_g, bn1_b, W2a, b2a, W2b, b2b, root2, bias2, bn2_g, bn2_b,
           Wfc1, bfc1, Wfc2, bfc2):
    e = edge_index.shape[1]
    src = edge_index[0]
    dst = edge_index[1]

    src_p = jnp.concatenate([src, jnp.zeros((E_PAD - e,), jnp.int32)])
    dst_p = jnp.concatenate([dst, jnp.full((E_PAD - e,), N, jnp.int32)])

    nh = W1a.shape[1]
    wb1 = W1b.reshape(nh * DN, 32)
    wb2 = W2b.reshape(nh * 32, 64)
    r1 = jnp.repeat(jnp.eye(nh, dtype=jnp.float32), DN, axis=1)
    t1 = jnp.tile(jnp.eye(DN, dtype=jnp.float32), (1, nh))
    r2 = jnp.repeat(jnp.eye(nh, dtype=jnp.float32), 32, axis=1)
    t2 = jnp.tile(jnp.eye(32, dtype=jnp.float32), (1, nh))

    p1 = []
    for h in range(2):
        xs1 = _sc_gather(x, src_p, DN, h)
        msg1 = _edge_call(edge_attr, xs1, W1a, b1a.reshape(1, -1), r1, t1,
                          wb1, b1b.reshape(DN, 32), DN, 16, h)
        p1.append(_sc_scatter_add(msg1, dst_p, 48, h))
    e1, st1 = _pre1_call(p1[0], p1[1], x, root1, bias1.reshape(1, -1))
    g1 = bn1_g.reshape(1, -1)
    bb1 = bn1_b.reshape(1, -1)

    p2 = []
    for h in range(2):
        xs2 = _sc_gather(e1, src_p, 48, h)
        msg2 = _edge_call(edge_attr, xs2, W2a, b2a.reshape(1, -1), r2, t2,
                          wb2, b2b.reshape(32, 64), 32, 0, h,
                          norm=(st1, g1, bb1))
        p2.append(_sc_scatter_add(msg2, dst_p, 64, h))

    e2, st2 = _pre2_call(p2[0], p2[1], e1, st1, g1, bb1, root2,
                         bias2.reshape(1, -1))
    out = _head_call(e2, st2, bn2_g.reshape(1, -1), bn2_b.reshape(1, -1),
                     Wfc1, bfc1.reshape(1, -1), Wfc2, bfc2.reshape(1, 1))
    return out.reshape(1)


# speedup vs baseline: 3.7453x; 1.0658x over previous
import functools

import jax
import jax.numpy as jnp
from jax import lax
from jax.experimental import pallas as pl
from jax.experimental.pallas import tpu as pltpu
from jax.experimental.pallas import tpu_sc as plsc

N = 20000
DN = 16
DE = 8
NC = 2
NS = 16
NW = NC * NS
CHUNK = 128
E_PAD = 81920
NCH = E_PAD // (NW * CHUNK)
GG = 2
N_PAD = 20096
RPS = N_PAD // NS
BE = 4096
LW = 128


def _elu(v):
    return jnp.where(v > 0, v, jnp.exp(v) - 1.0)


def _sc_mesh():
    return plsc.VectorSubcoreMesh(
        core_axis_name="c", subcore_axis_name="s",
        num_cores=NC, num_subcores=NS)


def _sc_gather(table, idx, dv, half):
    nch = NCH // 2
    e_half = E_PAD // 2

    def body(table_hbm, idx_hbm, out_hbm, idx_v, buf0, buf1,
             sem_g0, sem_g1, sem_w):
        wid = lax.axis_index("s") * NC + lax.axis_index("c")
        hbase = half * e_half + wid * nch * CHUNK
        base = wid * nch * CHUNK
        pltpu.sync_copy(idx_hbm.at[pl.ds(hbase, nch * CHUNK)], idx_v)

        bufs = [buf0, buf1]
        sems = [sem_g0, sem_g1]
        ng = nch // GG

        def fire(g):
            buf = bufs[g % 2]
            return [
                pltpu.async_copy(
                    table_hbm.at[idx_v.at[pl.ds((g * GG + k) * CHUNK, CHUNK)]],
                    buf.at[pl.ds(k * CHUNK, CHUNK)], sems[g % 2])
                for k in range(GG)
            ]

        wbs = [None, None]
        pend = fire(0)
        for g in range(ng):
            if wbs[(g + 1) % 2] is not None:
                wbs[(g + 1) % 2].wait()
                wbs[(g + 1) % 2] = None
            nxt = fire(g + 1) if g + 1 < ng else None
            for dsc in pend:
                dsc.wait()
            wbs[g % 2] = pltpu.async_copy(
                bufs[g % 2],
                out_hbm.at[pl.ds(base + g * GG * CHUNK, GG * CHUNK),
                           pl.ds(0, dv)], sem_w)
            pend = nxt
        for wb in wbs:
            if wb is not None:
                wb.wait()

    return pl.kernel(
        body,
        out_type=jax.ShapeDtypeStruct((e_half, LW), jnp.float32),
        mesh=_sc_mesh(),
        scratch_types=[
            pltpu.VMEM((nch * CHUNK,), jnp.int32),
            pltpu.VMEM((GG * CHUNK, dv), jnp.float32),
            pltpu.VMEM((GG * CHUNK, dv), jnp.float32),
            pltpu.SemaphoreType.DMA,
            pltpu.SemaphoreType.DMA,
            pltpu.SemaphoreType.DMA,
        ],
        compiler_params=pltpu.CompilerParams(use_tc_tiling_on_sc=False),
    )(table, idx)


def _sc_scatter_add(msg, idx, dv, half, init):
    nch = NCH // 2
    e_half = E_PAD // 2

    def body(msg_hbm, idx_hbm, init_hbm, out_hbm, idx_v0, idx_v1, msg_v0,
             msg_v1, acc_sh, sem_i0, sem_i1, sem_m0, sem_m1):
        c = lax.axis_index("c")
        s = lax.axis_index("s")
        wid = s * NC + c
        base = wid * nch * CHUNK
        ibase = half * e_half + base

        pltpu.sync_copy(init_hbm.at[c, pl.ds(s * RPS, RPS), pl.ds(0, dv)],
                        acc_sh.at[pl.ds(s * RPS, RPS)])
        plsc.subcore_barrier()

        bufs = [(idx_v0, msg_v0, sem_i0, sem_m0),
                (idx_v1, msg_v1, sem_i1, sem_m1)]

        def fire(j):
            iv, mv, si, sm = bufs[j % 2]
            di = pltpu.async_copy(idx_hbm.at[pl.ds(ibase + j * CHUNK, CHUNK)],
                                  iv, si)
            dm = pltpu.async_copy(
                msg_hbm.at[pl.ds(base + j * CHUNK, CHUNK), pl.ds(0, dv)],
                mv, sm)
            return di, dm

        pend = fire(0)
        for j in range(nch):
            nxt = fire(j + 1) if j + 1 < nch else None
            pend[0].wait()
            pend[1].wait()
            iv, mv = bufs[j % 2][0], bufs[j % 2][1]
            pltpu.sync_copy(mv, acc_sh.at[iv], add=True)
            pend = nxt

        plsc.subcore_barrier()
        pltpu.sync_copy(acc_sh.at[pl.ds(s * RPS, RPS)],
                        out_hbm.at[c, pl.ds(s * RPS, RPS), pl.ds(0, dv)])

    return pl.kernel(
        body,
        out_type=jax.ShapeDtypeStruct((NC, N_PAD, LW), jnp.float32),
        mesh=_sc_mesh(),
        scratch_types=[
            pltpu.VMEM((CHUNK,), jnp.int32),
            pltpu.VMEM((CHUNK,), jnp.int32),
            pltpu.VMEM((CHUNK, dv), jnp.float32),
            pltpu.VMEM((CHUNK, dv), jnp.float32),
            pltpu.VMEM_SHARED((N_PAD, dv), jnp.float32),
            pltpu.SemaphoreType.DMA,
            pltpu.SemaphoreType.DMA,
            pltpu.SemaphoreType.DMA,
            pltpu.SemaphoreType.DMA,
        ],
        compiler_params=pltpu.CompilerParams(use_tc_tiling_on_sc=False),
    )(msg, idx, init)


def _bn_apply(v, st_ref, g_ref, b_ref):
    mu = st_ref[0:1, :] * (1.0 / N)
    var = st_ref[1:2, :] * (1.0 / N) - mu * mu
    return _elu(g_ref[...] * (v - mu) * lax.rsqrt(var + 1e-5) + b_ref[...])


def _edge_body(cin, ones_cols, ea_ref, xs_ref, wa_ref, ba_ref, r_ref, t_ref,
               wb_ref, bb_ref, *rest):
    if len(rest) == 4:
        st_ref, g_ref, b_ref, out_ref = rest
        xs = _bn_apply(xs_ref[:, :cin], st_ref, g_ref, b_ref)
    else:
        (out_ref,) = rest
        xs = xs_ref[:, :cin]
    h = jnp.maximum(
        jnp.dot(ea_ref[...], wa_ref[...], preferred_element_type=jnp.float32)
        + ba_ref[...], 0.0)
    z = (jnp.dot(h, r_ref[...], preferred_element_type=jnp.float32)
         * jnp.dot(xs, t_ref[...], preferred_element_type=jnp.float32))
    msg = (jnp.dot(z, wb_ref[...], preferred_element_type=jnp.float32)
           + jnp.dot(xs, bb_ref[...], preferred_element_type=jnp.float32))
    if ones_cols:
        msg = jnp.concatenate(
            [msg, jnp.ones((msg.shape[0], ones_cols), jnp.float32)], axis=1)
    out_ref[:, :msg.shape[1]] = msg


def _edge_call(ea, xs, wa, ba, rmat, tmat, wb, bb, cin, ones_cols, half,
               norm=None):
    body = functools.partial(_edge_body, cin, ones_cols)
    args = [ea, xs, wa, ba, rmat, tmat, wb, bb]
    if norm is not None:
        args.extend(norm)
    e_half = E_PAD // 2
    nbh = e_half // BE
    off = half * nbh
    return pl.pallas_call(
        body,
        grid=(nbh,),
        in_specs=[
            pl.BlockSpec((BE, DE), lambda i: (i + off, 0)),
            pl.BlockSpec((BE, LW), lambda i: (i, 0)),
        ] + [pl.BlockSpec(a.shape, lambda i: (0, 0)) for a in args[2:]],
        out_specs=pl.BlockSpec((BE, LW), lambda i: (i, 0)),
        out_shape=jax.ShapeDtypeStruct((e_half, LW), jnp.float32),
    )(*args)


BN_BLK = 2000


def _pre1_body(part_ref, x_ref, root_ref, bias_ref, e1_ref, st_ref):
    i = pl.program_id(0)
    tot = part_ref[0, :, :48] + part_ref[1, :, :48]
    ssum = tot[:, :32]
    cnt = tot[:, 32:33]
    rc = 1.0 / jnp.maximum(cnt, 1.0)
    a = (ssum * rc
         + jnp.dot(x_ref[...], root_ref[...], preferred_element_type=jnp.float32)
         + bias_ref[...])
    e1 = _elu(a)
    e1_ref[:, :33] = jnp.concatenate([e1, rc], axis=1)

    @pl.when(i == 0)
    def _():
        st_ref[...] = jnp.zeros_like(st_ref)

    st_ref[0:1, :] += jnp.sum(e1, axis=0, keepdims=True)
    st_ref[1:2, :] += jnp.sum(e1 * e1, axis=0, keepdims=True)


def _pre1_call(part, x, root, bias):
    return pl.pallas_call(
        _pre1_body,
        grid=(N // BN_BLK,),
        in_specs=[
            pl.BlockSpec((2, BN_BLK, LW), lambda i: (0, i, 0)),
            pl.BlockSpec((BN_BLK, DN), lambda i: (i, 0)),
            pl.BlockSpec((DN, 32), lambda i: (0, 0)),
            pl.BlockSpec((1, 32), lambda i: (0, 0)),
        ],
        out_specs=(pl.BlockSpec((BN_BLK, 48), lambda i: (i, 0)),
                   pl.BlockSpec((8, 32), lambda i: (0, 0))),
        out_shape=(jax.ShapeDtypeStruct((N, 48), jnp.float32),
                   jax.ShapeDtypeStruct((8, 32), jnp.float32)),
    )(part, x, root, bias)


def _pre2_body(part_ref, e1_ref, st1_ref, g1_ref, b1_ref,
               root_ref, bias_ref, e2_ref, st_ref):
    i = pl.program_id(0)
    tot = part_ref[0, :, :64] + part_ref[1, :, :64]
    h1 = _bn_apply(e1_ref[:, :32], st1_ref, g1_ref, b1_ref)
    rc = e1_ref[:, 32:33]
    a = (tot * rc
         + jnp.dot(h1, root_ref[...], preferred_element_type=jnp.float32)
         + bias_ref[...])
    e2 = _elu(a)
    e2_ref[...] = e2

    @pl.when(i == 0)
    def _():
        st_ref[...] = jnp.zeros_like(st_ref)

    st_ref[0:1, :] += jnp.sum(e2, axis=0, keepdims=True)
    st_ref[1:2, :] += jnp.sum(e2 * e2, axis=0, keepdims=True)


def _pre2_call(part, e1, st1, g1, b1, root, bias):
    return pl.pallas_call(
        _pre2_body,
        grid=(N // BN_BLK,),
        in_specs=[
            pl.BlockSpec((2, BN_BLK, LW), lambda i: (0, i, 0)),
            pl.BlockSpec((BN_BLK, 48), lambda i: (i, 0)),
            pl.BlockSpec((8, 32), lambda i: (0, 0)),
            pl.BlockSpec((1, 32), lambda i: (0, 0)),
            pl.BlockSpec((1, 32), lambda i: (0, 0)),
            pl.BlockSpec((32, 64), lambda i: (0, 0)),
            pl.BlockSpec((1, 64), lambda i: (0, 0)),
        ],
        out_specs=(pl.BlockSpec((BN_BLK, 64), lambda i: (i, 0)),
                   pl.BlockSpec((8, 64), lambda i: (0, 0))),
        out_shape=(jax.ShapeDtypeStruct((N, 64), jnp.float32),
                   jax.ShapeDtypeStruct((8, 64), jnp.float32)),
    )(part, e1, st1, g1, b1, root, bias)


def _head_body(e2_ref, st_ref, g_ref, b_ref, wf1_ref, bf1_ref, wf2_ref,
               bf2_ref, out_ref):
    i = pl.program_id(0)
    h2 = _bn_apply(e2_ref[...], st_ref, g_ref, b_ref)
    f1 = _elu(jnp.dot(h2, wf1_ref[...], preferred_element_type=jnp.float32)
              + bf1_ref[...])
    f2 = _elu(jnp.dot(f1, wf2_ref[...], preferred_element_type=jnp.float32)
              + bf2_ref[...])

    @pl.when(i == 0)
    def _():
        out_ref[...] = jnp.zeros_like(out_ref)

    out_ref[...] += jnp.sum(f2).reshape(1, 1)


def _head_call(e2, st, g, b, wf1, bf1, wf2, bf2):
    return pl.pallas_call(
        _head_body,
        grid=(N // BN_BLK,),
        in_specs=[
            pl.BlockSpec((BN_BLK, 64), lambda i: (i, 0)),
            pl.BlockSpec((8, 64), lambda i: (0, 0)),
            pl.BlockSpec((1, 64), lambda i: (0, 0)),
            pl.BlockSpec((1, 64), lambda i: (0, 0)),
            pl.BlockSpec((64, 128), lambda i: (0, 0)),
            pl.BlockSpec((1, 128), lambda i: (0, 0)),
            pl.BlockSpec((128, 1), lambda i: (0, 0)),
            pl.BlockSpec((1, 1), lambda i: (0, 0)),
        ],
        out_specs=pl.BlockSpec((1, 1), lambda i: (0, 0)),
        out_shape=jax.ShapeDtypeStruct((1, 1), jnp.float32),
    )(e2, st, g, b, wf1, bf1, wf2, bf2)


def kernel(x, edge_index, edge_attr, W1a, b1a, W1b, b1b, root1, bias1,
           bn1_g, bn1_b, W2a, b2a, W2b, b2b, root2, bias2, bn2_g, bn2_b,
           Wfc1, bfc1, Wfc2, bfc2):
    e = edge_index.shape[1]
    src = edge_index[0]
    dst = edge_index[1]

    src_p = jnp.concatenate([src, jnp.zeros((E_PAD - e,), jnp.int32)])
    dst_p = jnp.concatenate([dst, jnp.full((E_PAD - e,), N, jnp.int32)])

    nh = W1a.shape[1]
    wb1 = W1b.reshape(nh * DN, 32)
    wb2 = W2b.reshape(nh * 32, 64)
    r1 = jnp.repeat(jnp.eye(nh, dtype=jnp.float32), DN, axis=1)
    t1 = jnp.tile(jnp.eye(DN, dtype=jnp.float32), (1, nh))
    r2 = jnp.repeat(jnp.eye(nh, dtype=jnp.float32), 32, axis=1)
    t2 = jnp.tile(jnp.eye(32, dtype=jnp.float32), (1, nh))

    zpart = jnp.zeros((NC, N_PAD, LW), jnp.float32)
    p1 = zpart
    for h in range(2):
        xs1 = _sc_gather(x, src_p, DN, h)
        msg1 = _edge_call(edge_attr, xs1, W1a, b1a.reshape(1, -1), r1, t1,
                          wb1, b1b.reshape(DN, 32), DN, 16, h)
        p1 = _sc_scatter_add(msg1, dst_p, 48, h, p1)
    e1, st1 = _pre1_call(p1, x, root1, bias1.reshape(1, -1))
    g1 = bn1_g.reshape(1, -1)
    bb1 = bn1_b.reshape(1, -1)

    p2 = zpart
    for h in range(2):
        xs2 = _sc_gather(e1, src_p, 48, h)
        msg2 = _edge_call(edge_attr, xs2, W2a, b2a.reshape(1, -1), r2, t2,
                          wb2, b2b.reshape(32, 64), 32, 0, h,
                          norm=(st1, g1, bb1))
        p2 = _sc_scatter_add(msg2, dst_p, 64, h, p2)

    e2, st2 = _pre2_call(p2, e1, st1, g1, bb1, root2,
                         bias2.reshape(1, -1))
    out = _head_call(e2, st2, bn2_g.reshape(1, -1), bn2_b.reshape(1, -1),
                     Wfc1, bfc1.reshape(1, -1), Wfc2, bfc2.reshape(1, 1))
    return out.reshape(1)
